# baseline (speedup 1.0000x reference)
import jax
import jax.numpy as jnp
from jax.experimental import pallas as pl
from jax.experimental.pallas import tpu as pltpu

STRIDE = 8
A = 3
NC = 80
C = 5 + NC

_BB = 8
_PL = 256


def _body(nb, nl, x_hbm, t_ref, o_ref, buf, add_buf, sem):
    bi = pl.program_id(0)
    li = pl.program_id(1)
    a = pl.program_id(2)
    k = bi * nl + li
    slot = jax.lax.rem(k, 2)
    i_off = li * (_PL // 64)

    def fetch(kk, ss):
        b2 = jax.lax.div(kk, nl)
        l2 = jax.lax.rem(kk, nl)
        pltpu.make_async_copy(
            x_hbm.at[pl.ds(b2 * _BB, _BB), pl.ds(l2 * _PL, _PL), :],
            buf.at[ss],
            sem.at[ss],
        ).start()

    @pl.when(a == 0)
    def _():
        @pl.when(k == 0)
        def _():
            fetch(0, 0)

        @pl.when(k + 1 < nb * nl)
        def _():
            fetch(k + 1, 1 - slot)

        add_i = t_ref[4, :, 0:C]
        add_j = t_ref[5, :, 0:C]
        p_i = jax.lax.broadcasted_iota(jnp.int32, (_PL, 1), 0)
        i_f = (p_i // 64 + i_off).astype(jnp.float32)
        j_f = (p_i % 64).astype(jnp.float32)
        add_buf[...] = i_f * add_i + j_f * add_j

        pltpu.make_async_copy(
            x_hbm.at[pl.ds(bi * _BB, _BB), pl.ds(li * _PL, _PL), :],
            buf.at[slot],
            sem.at[slot],
        ).wait()

    x_ref = buf.at[slot]

    def run(lo):
        sgn = t_ref[0, :, lo:lo + C]
        isexp = t_ref[1, :, lo:lo + C]
        invm = t_ref[2, :, lo:lo + C]
        em = t_ref[3, :, lo:lo + C]
        mask = isexp != 0.0
        add_term = add_buf[...]
        for b in range(_BB):
            v = x_ref[b, :, lo:lo + C]
            t = jnp.exp(v * sgn)
            den = t * invm + invm
            dec = jnp.where(mask, t * em, 1.0 / den) + add_term
            o_ref[:, b, :] = dec.T

    for aa in range(A):
        @pl.when(a == aa)
        def _(lo=aa * C):
            run(lo)


def kernel(x, anchors):
    B = x.shape[0]
    G = x.shape[2]
    P = G * G
    xt = jnp.transpose(x, (0, 2, 3, 1)).reshape(B, P, A * C)

    f32 = jnp.float32
    isexp_row = jnp.zeros((A, C), f32).at[:, 3:5].set(1.0)
    sgn_row = 2.0 * isexp_row - 1.0
    invm_row = jnp.ones((A, C), f32).at[:, 1:3].set(f32(1.0 / STRIDE))
    invm_row = invm_row.at[:, 3:5].set(1.0)
    em_row = jnp.zeros((A, C), f32).at[:, 3:5].set(anchors)
    addi_row = jnp.zeros((A, C), f32).at[:, 1].set(f32(STRIDE))
    addj_row = jnp.zeros((A, C), f32).at[:, 2].set(f32(STRIDE))
    tab = jnp.stack([sgn_row, isexp_row, invm_row, em_row,
                     addi_row, addj_row]).reshape(6, 1, A * C)

    nb = B // _BB
    nl = P // _PL
    import functools
    out = pl.pallas_call(
        functools.partial(_body, nb, nl),
        grid=(nb, nl, A),
        in_specs=[
            pl.BlockSpec(memory_space=pl.ANY),
            pl.BlockSpec((6, 1, A * C), lambda b, l, a: (0, 0, 0)),
        ],
        out_specs=pl.BlockSpec((C, _BB, _PL),
                               lambda b, l, a: (0, b, a * nl + l)),
        out_shape=jax.ShapeDtypeStruct((C, B, A * P), jnp.float32),
        scratch_shapes=[
            pltpu.VMEM((2, _BB, _PL, A * C), jnp.float32),
            pltpu.VMEM((_PL, C), jnp.float32),
            pltpu.SemaphoreType.DMA((2,)),
        ],
    )(xt, tab)
    return jnp.transpose(out, (1, 2, 0))


# speedup vs baseline: 1.1262x; 1.1262x over previous
import jax
import jax.numpy as jnp
from jax.experimental import pallas as pl
from jax.experimental.pallas import tpu as pltpu

STRIDE = 8
A = 3
NC = 80
C = 5 + NC

_BB = 8
_PL = 512


def _body(nb, nl, x_hbm, t_ref, o_ref, buf, add_buf, sem):
    bi = pl.program_id(0)
    li = pl.program_id(1)
    a = pl.program_id(2)
    k = bi * nl + li
    slot = jax.lax.rem(k, 3)
    i_off = li * (_PL // 64)

    def fetch(kk, ss):
        b2 = jax.lax.div(kk, nl)
        l2 = jax.lax.rem(kk, nl)
        pltpu.make_async_copy(
            x_hbm.at[pl.ds(b2 * _BB, _BB), pl.ds(l2 * _PL, _PL), :],
            buf.at[ss],
            sem.at[ss],
        ).start()

    @pl.when(a == 0)
    def _():
        @pl.when(k == 0)
        def _():
            fetch(0, 0)
            fetch(1, 1)

        @pl.when(k + 2 < nb * nl)
        def _():
            fetch(k + 2, jax.lax.rem(k + 2, 3))

        add_i = t_ref[4, :, 0:C]
        add_j = t_ref[5, :, 0:C]
        p_i = jax.lax.broadcasted_iota(jnp.int32, (_PL, 1), 0)
        i_f = (p_i // 64 + i_off).astype(jnp.float32)
        j_f = (p_i % 64).astype(jnp.float32)
        add_buf[...] = i_f * add_i + j_f * add_j

        pltpu.make_async_copy(
            x_hbm.at[pl.ds(bi * _BB, _BB), pl.ds(li * _PL, _PL), :],
            buf.at[slot],
            sem.at[slot],
        ).wait()

    x_ref = buf.at[slot]

    def run(lo):
        sgn = t_ref[0, :, lo:lo + C]
        isexp = t_ref[1, :, lo:lo + C]
        invm = t_ref[2, :, lo:lo + C]
        em = t_ref[3, :, lo:lo + C]
        mask = isexp != 0.0
        add_term = add_buf[...]
        for b in range(_BB):
            v = x_ref[b, :, lo:lo + C]
            t = jnp.exp(v * sgn)
            den = t * invm + invm
            dec = jnp.where(mask, t * em, 1.0 / den) + add_term
            o_ref[:, b, :] = dec.T

    for aa in range(A):
        @pl.when(a == aa)
        def _(lo=aa * C):
            run(lo)


def kernel(x, anchors):
    B = x.shape[0]
    G = x.shape[2]
    P = G * G
    xt = jnp.transpose(x, (0, 2, 3, 1)).reshape(B, P, A * C)

    f32 = jnp.float32
    isexp_row = jnp.zeros((A, C), f32).at[:, 3:5].set(1.0)
    sgn_row = 2.0 * isexp_row - 1.0
    invm_row = jnp.ones((A, C), f32).at[:, 1:3].set(f32(1.0 / STRIDE))
    invm_row = invm_row.at[:, 3:5].set(1.0)
    em_row = jnp.zeros((A, C), f32).at[:, 3:5].set(anchors)
    addi_row = jnp.zeros((A, C), f32).at[:, 1].set(f32(STRIDE))
    addj_row = jnp.zeros((A, C), f32).at[:, 2].set(f32(STRIDE))
    tab = jnp.stack([sgn_row, isexp_row, invm_row, em_row,
                     addi_row, addj_row]).reshape(6, 1, A * C)

    nb = B // _BB
    nl = P // _PL
    import functools
    out = pl.pallas_call(
        functools.partial(_body, nb, nl),
        grid=(nb, nl, A),
        in_specs=[
            pl.BlockSpec(memory_space=pl.ANY),
            pl.BlockSpec((6, 1, A * C), lambda b, l, a: (0, 0, 0)),
        ],
        out_specs=pl.BlockSpec((C, _BB, _PL),
                               lambda b, l, a: (0, b, a * nl + l)),
        out_shape=jax.ShapeDtypeStruct((C, B, A * P), jnp.float32),
        scratch_shapes=[
            pltpu.VMEM((3, _BB, _PL, A * C), jnp.float32),
            pltpu.VMEM((_PL, C), jnp.float32),
            pltpu.SemaphoreType.DMA((3,)),
        ],
    )(xt, tab)
    return jnp.transpose(out, (1, 2, 0))


# speedup vs baseline: 1.1793x; 1.0471x over previous
import jax
import jax.numpy as jnp
from jax.experimental import pallas as pl
from jax.experimental.pallas import tpu as pltpu

STRIDE = 8
A = 3
NC = 80
C = 5 + NC

_BB = 16
_PL = 512


def _body(nb, nl, x_hbm, t_ref, o_ref, buf, add_buf, sem):
    bi = pl.program_id(0)
    li = pl.program_id(1)
    a = pl.program_id(2)
    k = bi * nl + li
    slot = jax.lax.rem(k, 2)
    i_off = li * (_PL // 64)

    def fetch(kk, ss):
        b2 = jax.lax.div(kk, nl)
        l2 = jax.lax.rem(kk, nl)
        pltpu.make_async_copy(
            x_hbm.at[pl.ds(b2 * _BB, _BB), pl.ds(l2 * _PL, _PL), :],
            buf.at[ss],
            sem.at[ss],
        ).start()

    @pl.when(a == 0)
    def _():
        @pl.when(k == 0)
        def _():
            fetch(0, 0)

        @pl.when(k + 1 < nb * nl)
        def _():
            fetch(k + 1, 1 - slot)

        add_i = t_ref[4, :, 0:C]
        add_j = t_ref[5, :, 0:C]
        p_i = jax.lax.broadcasted_iota(jnp.int32, (_PL, 1), 0)
        i_f = (p_i // 64 + i_off).astype(jnp.float32)
        j_f = (p_i % 64).astype(jnp.float32)
        add_buf[...] = i_f * add_i + j_f * add_j

        pltpu.make_async_copy(
            x_hbm.at[pl.ds(bi * _BB, _BB), pl.ds(li * _PL, _PL), :],
            buf.at[slot],
            sem.at[slot],
        ).wait()

    x_ref = buf.at[slot]

    def run(lo):
        sgn = t_ref[0, :, lo:lo + C]
        isexp = t_ref[1, :, lo:lo + C]
        invm = t_ref[2, :, lo:lo + C]
        em = t_ref[3, :, lo:lo + C]
        mask = isexp != 0.0
        add_term = add_buf[...]
        for b in range(_BB):
            v = x_ref[b, :, lo:lo + C]
            t = jnp.exp(v * sgn)
            den = t * invm + invm
            dec = jnp.where(mask, t * em, 1.0 / den) + add_term
            o_ref[:, b, :] = dec.T

    for aa in range(A):
        @pl.when(a == aa)
        def _(lo=aa * C):
            run(lo)


def kernel(x, anchors):
    B = x.shape[0]
    G = x.shape[2]
    P = G * G
    xt = jnp.transpose(x, (0, 2, 3, 1)).reshape(B, P, A * C)

    f32 = jnp.float32
    isexp_row = jnp.zeros((A, C), f32).at[:, 3:5].set(1.0)
    sgn_row = 2.0 * isexp_row - 1.0
    invm_row = jnp.ones((A, C), f32).at[:, 1:3].set(f32(1.0 / STRIDE))
    invm_row = invm_row.at[:, 3:5].set(1.0)
    em_row = jnp.zeros((A, C), f32).at[:, 3:5].set(anchors)
    addi_row = jnp.zeros((A, C), f32).at[:, 1].set(f32(STRIDE))
    addj_row = jnp.zeros((A, C), f32).at[:, 2].set(f32(STRIDE))
    tab = jnp.stack([sgn_row, isexp_row, invm_row, em_row,
                     addi_row, addj_row]).reshape(6, 1, A * C)

    nb = B // _BB
    nl = P // _PL
    import functools
    out = pl.pallas_call(
        functools.partial(_body, nb, nl),
        grid=(nb, nl, A),
        in_specs=[
            pl.BlockSpec(memory_space=pl.ANY),
            pl.BlockSpec((6, 1, A * C), lambda b, l, a: (0, 0, 0)),
        ],
        out_specs=pl.BlockSpec((C, _BB, _PL),
                               lambda b, l, a: (0, b, a * nl + l)),
        out_shape=jax.ShapeDtypeStruct((C, B, A * P), jnp.float32),
        scratch_shapes=[
            pltpu.VMEM((2, _BB, _PL, A * C), jnp.float32),
            pltpu.VMEM((_PL, C), jnp.float32),
            pltpu.SemaphoreType.DMA((2,)),
        ],
    )(xt, tab)
    return jnp.transpose(out, (1, 2, 0))
